# Initial kernel scaffold; baseline (speedup 1.0000x reference)
import jax
import jax.numpy as jnp
from jax.experimental import pallas as pl


def kernel(x, batch, W1, b1, a1, W2, b2, a2, bn_g, bn_b, a_act, Wm0, bm0, Wm1, bm1, Wm2, bm2, Wm3, bm3, Wf1, bf1, Wf2, bf2):
    raise NotImplementedError("write your pallas kernel here")


# speedup vs baseline: 8.3921x; 8.3921x over previous
import functools

import jax
import jax.numpy as jnp
from jax import lax
from jax.experimental import pallas as pl
from jax.experimental.pallas import tpu as pltpu
from jax.experimental.pallas import tpu_sc as plsc

N = 10000
F = 128
K = 6
G = 16
H = 5 * F

BQ = 500
BK = 1000
NQB = N // BQ
NKB = N // BK

NP = 10240
NW = 32
CW = NP // NW
CH = 80
NCH = CW // CH


def _knn_body(lo_ref, hi_ref, x_ref, x2c_ref, x2r_ref, s_ref, e_ref, out_ref):
    i = pl.program_id(0)
    xq = x_ref[pl.ds(i * BQ, BQ), :]
    xq2 = x2c_ref[pl.ds(i * BQ, BQ), :]
    sq = s_ref[pl.ds(i * BQ, BQ), :]
    eq = e_ref[pl.ds(i * BQ, BQ), :]
    row_idx = i * BQ + lax.broadcasted_iota(jnp.int32, (BQ, 1), 0)
    big = jnp.int32(2 ** 30)
    inf = jnp.float32(jnp.inf)

    def kblock(j, carry):
        vs = list(carry[0])
        ids = list(carry[1])
        xk = x_ref[pl.ds(j * BK, BK), :]
        xk2 = x2r_ref[j]
        dots = lax.dot_general(xq, xk, (((1,), (1,)), ((), ())),
                               preferred_element_type=jnp.float32)
        d = xq2 - 2.0 * dots + xk2
        kidx = j * BK + lax.broadcasted_iota(jnp.int32, (BQ, BK), 1)
        valid = (kidx >= sq) & (kidx < eq) & (kidx != row_idx)
        d = jnp.where(valid, d, inf)
        for _ in range(K):
            m = jnp.min(d, axis=1, keepdims=True)
            cand = jnp.min(jnp.where(d == m, kidx, big), axis=1, keepdims=True)
            d = jnp.where(kidx == cand, inf, d)
            c, ci = m, cand
            for t in range(K):
                sw = c < vs[t]
                vs[t], c = jnp.where(sw, c, vs[t]), jnp.where(sw, vs[t], c)
                ids[t], ci = jnp.where(sw, ci, ids[t]), jnp.where(sw, ids[t], ci)
        return tuple(vs), tuple(ids)

    init = (tuple(jnp.full((BQ, 1), inf, jnp.float32) for _ in range(K)),
            tuple(jnp.full((BQ, 1), t, jnp.int32) for t in range(K)))
    _, ids = lax.fori_loop(lo_ref[i], hi_ref[i], kblock, init)
    pad = jnp.zeros((BQ, 1), jnp.int32)
    out_ref[pl.ds(i * BQ, BQ), :] = jnp.concatenate(list(ids) + [pad, pad],
                                                    axis=1)


def _knn(lo_kb, hi_kb, x, x2_col, x2_rows, s_col, e_col):
    grid_spec = pltpu.PrefetchScalarGridSpec(
        num_scalar_prefetch=2,
        grid=(NQB,),
        in_specs=[
            pl.BlockSpec((N, F), lambda i, lo, hi: (0, 0)),
            pl.BlockSpec((N, 1), lambda i, lo, hi: (0, 0)),
            pl.BlockSpec((NKB, 1, BK), lambda i, lo, hi: (0, 0, 0)),
            pl.BlockSpec((N, 1), lambda i, lo, hi: (0, 0)),
            pl.BlockSpec((N, 1), lambda i, lo, hi: (0, 0)),
        ],
        out_specs=pl.BlockSpec((N, 8), lambda i, lo, hi: (0, 0)),
    )
    return pl.pallas_call(
        _knn_body,
        grid_spec=grid_spec,
        out_shape=jax.ShapeDtypeStruct((N, 8), jnp.int32),
        compiler_params=pltpu.CompilerParams(
            dimension_semantics=("arbitrary",)),
    )(lo_kb, hi_kb, x, x2_col, x2_rows, s_col, e_col)


def _mlp_body(x_ref, w1_ref, b1_ref, a1_ref, w2_ref, b2_ref, a2_ref,
              g_ref, bb_ref, out_ref):
    h = jnp.dot(x_ref[...], w1_ref[...],
                preferred_element_type=jnp.float32) + b1_ref[...]
    h = jnp.where(h > 0, h, a1_ref[...] * h)
    h = jnp.dot(h, w2_ref[...],
                preferred_element_type=jnp.float32) + b2_ref[...]
    h = jnp.where(h > 0, h, a2_ref[...] * h)
    mu = jnp.mean(h, axis=0, keepdims=True)
    dv = h - mu
    var = jnp.mean(dv * dv, axis=0, keepdims=True)
    out_ref[...] = dv / jnp.sqrt(var + 1e-5) * g_ref[...] + bb_ref[...]


def _mlp(x, w1, b1, a1, w2, b2, a2, g, bb):
    return pl.pallas_call(
        _mlp_body,
        out_shape=jax.ShapeDtypeStruct((N, F), jnp.float32),
    )(x, w1, b1, a1, w2, b2, a2, g, bb)


def _gather_sum(h_pad, idx6):
    mesh = plsc.VectorSubcoreMesh(core_axis_name="c", subcore_axis_name="s")
    nc = plsc.get_sparse_core_info().num_cores

    @functools.partial(
        pl.kernel,
        out_type=jax.ShapeDtypeStruct((NP, F), jnp.float32),
        mesh=mesh,
        scratch_types=[
            pltpu.VMEM((K * CW,), jnp.int32),
            pltpu.VMEM((K, CH, F), jnp.float32),
            pltpu.VMEM((CW, F), jnp.float32),
            pltpu.SemaphoreType.DMA,
        ],
    )
    def gs(h_hbm, idx_hbm, out_hbm, idx_v, buf, outv, sem):
        wid = lax.axis_index("s") * nc + lax.axis_index("c")
        base = wid * CW
        for j in range(K):
            pltpu.sync_copy(idx_hbm.at[pl.ds(j * NP + base, CW)],
                            idx_v.at[pl.ds(j * CW, CW)])
        for c in range(NCH):
            cops = [
                pltpu.make_async_copy(
                    h_hbm.at[idx_v.at[pl.ds(j * CW + c * CH, CH)]],
                    buf.at[j], sem)
                for j in range(K)
            ]
            for cp in cops:
                cp.start()
            for cp in cops:
                cp.wait()

            def rowloop(r, _, c=c):
                for l in range(F // 16):
                    sl = pl.ds(l * 16, 16)
                    v = (buf[0, r, sl] + buf[1, r, sl] + buf[2, r, sl]
                         + buf[3, r, sl] + buf[4, r, sl] + buf[5, r, sl])
                    outv[c * CH + r, sl] = v
                return 0

            lax.fori_loop(0, CH, rowloop, 0, unroll=2)
        pltpu.sync_copy(outv, out_hbm.at[pl.ds(base, CW)])

    return gs(h_pad, idx6)


def _layer_body(s_ref, h_ref, wm_ref, bm_ref, a_ref, out_ref):
    z = jnp.dot(s_ref[...], wm_ref[...], preferred_element_type=jnp.float32)
    z = z + jnp.float32(K) * bm_ref[...] + h_ref[...]
    hn = jnp.where(z > 0, z, a_ref[...] * z)
    rows = lax.broadcasted_iota(jnp.int32, (NP, 1), 0)
    out_ref[...] = jnp.where(rows < N, hn, 0.0)


def _layer(s, h, wm, bm, a):
    return pl.pallas_call(
        _layer_body,
        out_shape=jax.ShapeDtypeStruct((NP, F), jnp.float32),
    )(s, h, wm, bm, a)


def _final_body(h0_ref, h1_ref, h2_ref, h3_ref, h4_ref, b_ref,
                wf1_ref, bf1_ref, wf2_ref, bf2_ref, out_ref):
    bvec = b_ref[...]
    gid = lax.broadcasted_iota(jnp.int32, (NP, G), 1)
    oh = (bvec == gid).astype(jnp.float32)
    pooled = [
        lax.dot_general(oh, hr[...], (((0,), (0,)), ((), ())),
                        preferred_element_type=jnp.float32)
        for hr in (h0_ref, h1_ref, h2_ref, h3_ref, h4_ref)
    ]
    p = jnp.concatenate(pooled, axis=1)
    t = jnp.dot(p, wf1_ref[...],
                preferred_element_type=jnp.float32) + bf1_ref[...]
    t = jnp.maximum(t, 0.0)
    out_ref[...] = (jnp.sum(t * wf2_ref[...], axis=1, keepdims=True)
                    + bf2_ref[...])


def _final(hs, bcol, wf1, bf1, wf2t, bf2):
    return pl.pallas_call(
        _final_body,
        out_shape=jax.ShapeDtypeStruct((G, 1), jnp.float32),
    )(*hs, bcol, wf1, bf1, wf2t, bf2)


def kernel(x, batch, W1, b1, a1, W2, b2, a2, bn_g, bn_b, a_act,
           Wm0, bm0, Wm1, bm1, Wm2, bm2, Wm3, bm3, Wf1, bf1, Wf2, bf2):
    batch = batch.astype(jnp.int32)
    gids = jnp.arange(G, dtype=jnp.int32)
    starts = jnp.sum(batch[None, :] < gids[:, None], axis=1).astype(jnp.int32)
    ends = jnp.sum(batch[None, :] <= gids[:, None], axis=1).astype(jnp.int32)
    s_col = starts[batch][:, None]
    e_col = ends[batch][:, None]
    qs = jnp.arange(NQB, dtype=jnp.int32)
    lo_kb = s_col[qs * BQ, 0] // BK
    hi_kb = (e_col[qs * BQ + BQ - 1, 0] + BK - 1) // BK

    x2 = jnp.sum(x * x, axis=1)
    idx8 = _knn(lo_kb, hi_kb, x, x2[:, None], x2.reshape(NKB, 1, BK),
                s_col, e_col)
    idx6 = jnp.pad(idx8[:, :K].T, ((0, 0), (0, NP - N))).reshape(-1)

    h = _mlp(x, W1, b1[None, :], a1[None, :], W2, b2[None, :], a2[None, :],
             bn_g[None, :], bn_b[None, :])
    h = jnp.pad(h, ((0, NP - N), (0, 0)))

    hs = [h]
    for wm, bm in ((Wm0, bm0), (Wm1, bm1), (Wm2, bm2), (Wm3, bm3)):
        s = _gather_sum(h, idx6)
        h = _layer(s, h, wm, bm[None, :], a_act[None, :])
        hs.append(h)

    bcol = jnp.pad(batch, (0, NP - N), constant_values=G)[:, None]
    return _final(hs, bcol, Wf1, bf1[None, :], Wf2.reshape(1, H),
                  bf2[None, :])


# speedup vs baseline: 8.5998x; 1.0248x over previous
import functools

import jax
import jax.numpy as jnp
from jax import lax
from jax.experimental import pallas as pl
from jax.experimental.pallas import tpu as pltpu
from jax.experimental.pallas import tpu_sc as plsc

N = 10000
F = 128
K = 6
G = 16
H = 5 * F

BQ = 500
BK = 1000
NQB = N // BQ
NKB = N // BK

NP = 10240
NW = 32
CW = NP // NW
CH = 40
NCH = CW // CH


def _knn_body(lo_ref, hi_ref, x_ref, x2c_ref, x2r_ref, s_ref, e_ref, out_ref):
    i = pl.program_id(0)
    xq = x_ref[pl.ds(i * BQ, BQ), :]
    xq2 = x2c_ref[pl.ds(i * BQ, BQ), :]
    sq = s_ref[pl.ds(i * BQ, BQ), :]
    eq = e_ref[pl.ds(i * BQ, BQ), :]
    row_idx = i * BQ + lax.broadcasted_iota(jnp.int32, (BQ, 1), 0)
    big = jnp.int32(2 ** 30)
    inf = jnp.float32(jnp.inf)

    def kblock(j, carry):
        vs = list(carry[0])
        ids = list(carry[1])
        xk = x_ref[pl.ds(j * BK, BK), :]
        xk2 = x2r_ref[j]
        dots = lax.dot_general(xq, xk, (((1,), (1,)), ((), ())),
                               preferred_element_type=jnp.float32)
        d = xq2 - 2.0 * dots + xk2
        kidx = j * BK + lax.broadcasted_iota(jnp.int32, (BQ, BK), 1)
        valid = (kidx >= sq) & (kidx < eq) & (kidx != row_idx)
        d = jnp.where(valid, d, inf)
        for _ in range(K):
            m = jnp.min(d, axis=1, keepdims=True)
            cand = jnp.min(jnp.where(d == m, kidx, big), axis=1, keepdims=True)
            d = jnp.where(kidx == cand, inf, d)
            c, ci = m, cand
            for t in range(K):
                sw = c < vs[t]
                vs[t], c = jnp.where(sw, c, vs[t]), jnp.where(sw, vs[t], c)
                ids[t], ci = jnp.where(sw, ci, ids[t]), jnp.where(sw, ids[t], ci)
        return tuple(vs), tuple(ids)

    init = (tuple(jnp.full((BQ, 1), inf, jnp.float32) for _ in range(K)),
            tuple(jnp.full((BQ, 1), t, jnp.int32) for t in range(K)))
    _, ids = lax.fori_loop(lo_ref[i], hi_ref[i], kblock, init)
    pad = jnp.zeros((BQ, 1), jnp.int32)
    out_ref[pl.ds(i * BQ, BQ), :] = jnp.concatenate(list(ids) + [pad, pad],
                                                    axis=1)


def _knn(lo_kb, hi_kb, x, x2_col, x2_rows, s_col, e_col):
    grid_spec = pltpu.PrefetchScalarGridSpec(
        num_scalar_prefetch=2,
        grid=(NQB,),
        in_specs=[
            pl.BlockSpec((N, F), lambda i, lo, hi: (0, 0)),
            pl.BlockSpec((N, 1), lambda i, lo, hi: (0, 0)),
            pl.BlockSpec((NKB, 1, BK), lambda i, lo, hi: (0, 0, 0)),
            pl.BlockSpec((N, 1), lambda i, lo, hi: (0, 0)),
            pl.BlockSpec((N, 1), lambda i, lo, hi: (0, 0)),
        ],
        out_specs=pl.BlockSpec((N, 8), lambda i, lo, hi: (0, 0)),
    )
    return pl.pallas_call(
        _knn_body,
        grid_spec=grid_spec,
        out_shape=jax.ShapeDtypeStruct((N, 8), jnp.int32),
        compiler_params=pltpu.CompilerParams(
            dimension_semantics=("arbitrary",)),
    )(lo_kb, hi_kb, x, x2_col, x2_rows, s_col, e_col)


def _mlp_body(x_ref, w1_ref, b1_ref, a1_ref, w2_ref, b2_ref, a2_ref,
              g_ref, bb_ref, out_ref):
    h = jnp.dot(x_ref[...], w1_ref[...],
                preferred_element_type=jnp.float32) + b1_ref[...]
    h = jnp.where(h > 0, h, a1_ref[...] * h)
    h = jnp.dot(h, w2_ref[...],
                preferred_element_type=jnp.float32) + b2_ref[...]
    h = jnp.where(h > 0, h, a2_ref[...] * h)
    mu = jnp.mean(h, axis=0, keepdims=True)
    dv = h - mu
    var = jnp.mean(dv * dv, axis=0, keepdims=True)
    out_ref[...] = dv / jnp.sqrt(var + 1e-5) * g_ref[...] + bb_ref[...]


def _mlp(x, w1, b1, a1, w2, b2, a2, g, bb):
    return pl.pallas_call(
        _mlp_body,
        out_shape=jax.ShapeDtypeStruct((N, F), jnp.float32),
    )(x, w1, b1, a1, w2, b2, a2, g, bb)


def _gather_sum(h_pad, idx6):
    mesh = plsc.VectorSubcoreMesh(core_axis_name="c", subcore_axis_name="s")
    nc = plsc.get_sparse_core_info().num_cores

    @functools.partial(
        pl.kernel,
        out_type=jax.ShapeDtypeStruct((NP, F), jnp.float32),
        mesh=mesh,
        scratch_types=[
            pltpu.VMEM((K * CW,), jnp.int32),
            pltpu.VMEM((2, K, CH, F), jnp.float32),
            pltpu.VMEM((CW, F), jnp.float32),
            pltpu.SemaphoreType.DMA,
            pltpu.SemaphoreType.DMA,
        ],
    )
    def gs(h_hbm, idx_hbm, out_hbm, idx_v, buf, outv, sem0, sem1):
        wid = lax.axis_index("s") * nc + lax.axis_index("c")
        base = wid * CW
        sems = (sem0, sem1)
        for j in range(K):
            pltpu.sync_copy(idx_hbm.at[pl.ds(j * NP + base, CW)],
                            idx_v.at[pl.ds(j * CW, CW)])

        def copies(c):
            b = c % 2
            return [
                pltpu.make_async_copy(
                    h_hbm.at[idx_v.at[pl.ds(j * CW + c * CH, CH)]],
                    buf.at[b, j], sems[b])
                for j in range(K)
            ]

        for cp in copies(0):
            cp.start()
        for c in range(NCH):
            b = c % 2
            if c + 1 < NCH:
                for cp in copies(c + 1):
                    cp.start()
            for cp in copies(c):
                cp.wait()

            def rowloop(r, _, c=c, b=b):
                for l in range(F // 16):
                    sl = pl.ds(l * 16, 16)
                    v = (buf[b, 0, r, sl] + buf[b, 1, r, sl]
                         + buf[b, 2, r, sl] + buf[b, 3, r, sl]
                         + buf[b, 4, r, sl] + buf[b, 5, r, sl])
                    outv[c * CH + r, sl] = v
                return 0

            lax.fori_loop(0, CH, rowloop, 0, unroll=4)
        pltpu.sync_copy(outv, out_hbm.at[pl.ds(base, CW)])

    return gs(h_pad, idx6)


def _layer_body(s_ref, h_ref, wm_ref, bm_ref, a_ref, out_ref):
    z = jnp.dot(s_ref[...], wm_ref[...], preferred_element_type=jnp.float32)
    z = z + jnp.float32(K) * bm_ref[...] + h_ref[...]
    hn = jnp.where(z > 0, z, a_ref[...] * z)
    rows = lax.broadcasted_iota(jnp.int32, (NP, 1), 0)
    out_ref[...] = jnp.where(rows < N, hn, 0.0)


def _layer(s, h, wm, bm, a):
    return pl.pallas_call(
        _layer_body,
        out_shape=jax.ShapeDtypeStruct((NP, F), jnp.float32),
    )(s, h, wm, bm, a)


def _final_body(h0_ref, h1_ref, h2_ref, h3_ref, h4_ref, b_ref,
                wf1_ref, bf1_ref, wf2_ref, bf2_ref, out_ref):
    bvec = b_ref[...]
    gid = lax.broadcasted_iota(jnp.int32, (NP, G), 1)
    oh = (bvec == gid).astype(jnp.float32)
    pooled = [
        lax.dot_general(oh, hr[...], (((0,), (0,)), ((), ())),
                        preferred_element_type=jnp.float32)
        for hr in (h0_ref, h1_ref, h2_ref, h3_ref, h4_ref)
    ]
    p = jnp.concatenate(pooled, axis=1)
    t = jnp.dot(p, wf1_ref[...],
                preferred_element_type=jnp.float32) + bf1_ref[...]
    t = jnp.maximum(t, 0.0)
    out_ref[...] = (jnp.sum(t * wf2_ref[...], axis=1, keepdims=True)
                    + bf2_ref[...])


def _final(hs, bcol, wf1, bf1, wf2t, bf2):
    return pl.pallas_call(
        _final_body,
        out_shape=jax.ShapeDtypeStruct((G, 1), jnp.float32),
    )(*hs, bcol, wf1, bf1, wf2t, bf2)


def kernel(x, batch, W1, b1, a1, W2, b2, a2, bn_g, bn_b, a_act,
           Wm0, bm0, Wm1, bm1, Wm2, bm2, Wm3, bm3, Wf1, bf1, Wf2, bf2):
    batch = batch.astype(jnp.int32)
    gids = jnp.arange(G, dtype=jnp.int32)
    starts = jnp.sum(batch[None, :] < gids[:, None], axis=1).astype(jnp.int32)
    ends = jnp.sum(batch[None, :] <= gids[:, None], axis=1).astype(jnp.int32)
    s_col = starts[batch][:, None]
    e_col = ends[batch][:, None]
    qs = jnp.arange(NQB, dtype=jnp.int32)
    lo_kb = s_col[qs * BQ, 0] // BK
    hi_kb = (e_col[qs * BQ + BQ - 1, 0] + BK - 1) // BK

    x2 = jnp.sum(x * x, axis=1)
    idx8 = _knn(lo_kb, hi_kb, x, x2[:, None], x2.reshape(NKB, 1, BK),
                s_col, e_col)
    idx6 = jnp.pad(idx8[:, :K].T, ((0, 0), (0, NP - N))).reshape(-1)

    h = _mlp(x, W1, b1[None, :], a1[None, :], W2, b2[None, :], a2[None, :],
             bn_g[None, :], bn_b[None, :])
    h = jnp.pad(h, ((0, NP - N), (0, 0)))

    hs = [h]
    for wm, bm in ((Wm0, bm0), (Wm1, bm1), (Wm2, bm2), (Wm3, bm3)):
        s = _gather_sum(h, idx6)
        h = _layer(s, h, wm, bm[None, :], a_act[None, :])
        hs.append(h)

    bcol = jnp.pad(batch, (0, NP - N), constant_values=G)[:, None]
    return _final(hs, bcol, Wf1, bf1[None, :], Wf2.reshape(1, H),
                  bf2[None, :])
